# Initial kernel scaffold; baseline (speedup 1.0000x reference)
import jax
import jax.numpy as jnp
from jax.experimental import pallas as pl


def kernel(x, ei, W_lin, b_lin, W1, b1, W2, b2):
    raise NotImplementedError("write your pallas kernel here")


# speedup vs baseline: 28.8124x; 28.8124x over previous
import functools

import jax
import jax.numpy as jnp
from jax import lax
from jax.experimental import pallas as pl
from jax.experimental.pallas import tpu as pltpu
from jax.experimental.pallas import tpu_sc as plsc

N_PAD = 10240
CHUNK = 128
NWORK = 32
BLK = 1024


def _enc_body(x_ref, wl_ref, bl_ref, w1_ref, o_ref):
    h0 = jnp.dot(x_ref[...], wl_ref[...].T, preferred_element_type=jnp.float32)
    h0 = jnp.maximum(h0 + bl_ref[...], 0.0)
    o_ref[...] = jnp.dot(h0, w1_ref[...].T, preferred_element_type=jnp.float32)


def _scale_body(h1_ref, degp_ref, hs_ref, dinv_ref):
    deg = degp_ref[0] + degp_ref[1] + 1.0
    dinv = lax.rsqrt(deg)
    hs_ref[...] = h1_ref[...] * dinv[:, :1]
    dinv_ref[...] = dinv


def _layer1_body(agg_ref, hs_ref, dinv_ref, b1_ref, w2_ref, o_ref):
    dinv = dinv_ref[:, :1]
    z = (agg_ref[0] + agg_ref[1] + hs_ref[...]) * dinv + b1_ref[...]
    z = jnp.maximum(z, 0.0)
    h2 = jnp.dot(z, w2_ref[...].T, preferred_element_type=jnp.float32)
    o_ref[...] = h2 * dinv


def _final_body(agg_ref, hs2_ref, dinv_ref, b2_ref, o_ref):
    dinv = dinv_ref[:, :1]
    o_ref[...] = (agg_ref[0] + agg_ref[1] + hs2_ref[...]) * dinv + b2_ref[...]


_SC_PARAMS = pltpu.CompilerParams(use_tc_tiling_on_sc=False)


def _make_deg_kernel(cpw):
    mesh = plsc.VectorSubcoreMesh(core_axis_name="c", subcore_axis_name="s")

    @functools.partial(
        pl.kernel,
        out_type=jax.ShapeDtypeStruct((2, N_PAD, 8), jnp.float32),
        mesh=mesh,
        compiler_params=_SC_PARAMS,
        scratch_types=[
            pltpu.VMEM((cpw, CHUNK), jnp.int32),
            pltpu.VMEM((CHUNK, 8), jnp.float32),
            pltpu.VMEM_SHARED((N_PAD, 8), jnp.float32),
        ],
    )
    def deg_kernel(dst_hbm, ones_hbm, zeros_hbm, out_hbm, idx_v, ones_v, acc_s):
        c = lax.axis_index("c")
        s = lax.axis_index("s")
        wid = s * 2 + c
        rpt = N_PAD // 16
        r0 = s * rpt
        pltpu.sync_copy(zeros_hbm.at[pl.ds(r0, rpt)], acc_s.at[pl.ds(r0, rpt)])
        pltpu.sync_copy(ones_hbm, ones_v)
        pltpu.sync_copy(dst_hbm.at[wid], idx_v)
        plsc.subcore_barrier()

        def body(j, carry):
            pltpu.sync_copy(ones_v, acc_s.at[idx_v.at[j]], add=True)
            return carry

        lax.fori_loop(0, cpw, body, 0)
        plsc.subcore_barrier()
        pltpu.sync_copy(acc_s.at[pl.ds(r0, rpt)], out_hbm.at[c, pl.ds(r0, rpt)])

    return deg_kernel


def _make_edge_kernel(cpw, d):
    mesh = plsc.VectorSubcoreMesh(core_axis_name="c", subcore_axis_name="s")

    @functools.partial(
        pl.kernel,
        out_type=jax.ShapeDtypeStruct((2, N_PAD, d), jnp.float32),
        mesh=mesh,
        compiler_params=_SC_PARAMS,
        scratch_types=[
            pltpu.VMEM((cpw, CHUNK), jnp.int32),
            pltpu.VMEM((cpw, CHUNK), jnp.int32),
            pltpu.VMEM((CHUNK, d), jnp.float32),
            pltpu.VMEM_SHARED((N_PAD, d), jnp.float32),
            pltpu.SemaphoreType.DMA,
        ],
    )
    def edge_kernel(src_hbm, dst_hbm, table_hbm, zeros_hbm, out_hbm,
                    src_v, dst_v, rows_v, acc_s, sem):
        c = lax.axis_index("c")
        s = lax.axis_index("s")
        wid = s * 2 + c
        rpt = N_PAD // 16
        r0 = s * rpt
        pltpu.sync_copy(zeros_hbm.at[pl.ds(r0, rpt)], acc_s.at[pl.ds(r0, rpt)])
        pltpu.sync_copy(src_hbm.at[wid], src_v)
        pltpu.sync_copy(dst_hbm.at[wid], dst_v)
        plsc.subcore_barrier()

        def body(j, carry):
            pltpu.async_copy(table_hbm.at[src_v.at[j]], rows_v, sem).wait()
            pltpu.sync_copy(rows_v, acc_s.at[dst_v.at[j]], add=True)
            return carry

        lax.fori_loop(0, cpw, body, 0)
        plsc.subcore_barrier()
        pltpu.sync_copy(acc_s.at[pl.ds(r0, rpt)], out_hbm.at[c, pl.ds(r0, rpt)])

    return edge_kernel


def kernel(x, ei, W_lin, b_lin, W1, b1, W2, b2):
    n, feat = x.shape
    hid = W1.shape[0]
    emb = W2.shape[0]
    e = ei.shape[1]
    cpw = -(-e // (NWORK * CHUNK))
    ep = NWORK * cpw * CHUNK

    xp = jnp.zeros((N_PAD, feat), jnp.float32).at[:n].set(x)
    pad = jnp.full((ep - e,), n, dtype=jnp.int32)
    srcp = jnp.concatenate([ei[0], pad]).reshape(NWORK, cpw, CHUNK)
    dstp = jnp.concatenate([ei[1], pad]).reshape(NWORK, cpw, CHUNK)
    ones8 = jnp.ones((CHUNK, 8), jnp.float32)
    zeros8 = jnp.zeros((N_PAD, 8), jnp.float32)
    zeros_h = jnp.zeros((N_PAD, hid), jnp.float32)
    zeros_e = jnp.zeros((N_PAD, emb), jnp.float32)

    grid = (N_PAD // BLK,)

    h1 = pl.pallas_call(
        _enc_body,
        grid=grid,
        in_specs=[
            pl.BlockSpec((BLK, feat), lambda i: (i, 0)),
            pl.BlockSpec((hid, feat), lambda i: (0, 0)),
            pl.BlockSpec((1, hid), lambda i: (0, 0)),
            pl.BlockSpec((hid, hid), lambda i: (0, 0)),
        ],
        out_specs=pl.BlockSpec((BLK, hid), lambda i: (i, 0)),
        out_shape=jax.ShapeDtypeStruct((N_PAD, hid), jnp.float32),
    )(xp, W_lin, b_lin.reshape(1, hid), W1)

    degp = _make_deg_kernel(cpw)(dstp, ones8, zeros8)

    hs1, dinv8 = pl.pallas_call(
        _scale_body,
        grid=grid,
        in_specs=[
            pl.BlockSpec((BLK, hid), lambda i: (i, 0)),
            pl.BlockSpec((2, BLK, 8), lambda i: (0, i, 0)),
        ],
        out_specs=[
            pl.BlockSpec((BLK, hid), lambda i: (i, 0)),
            pl.BlockSpec((BLK, 8), lambda i: (i, 0)),
        ],
        out_shape=[
            jax.ShapeDtypeStruct((N_PAD, hid), jnp.float32),
            jax.ShapeDtypeStruct((N_PAD, 8), jnp.float32),
        ],
    )(h1, degp)

    agg1 = _make_edge_kernel(cpw, hid)(srcp, dstp, hs1, zeros_h)

    hs2 = pl.pallas_call(
        _layer1_body,
        grid=grid,
        in_specs=[
            pl.BlockSpec((2, BLK, hid), lambda i: (0, i, 0)),
            pl.BlockSpec((BLK, hid), lambda i: (i, 0)),
            pl.BlockSpec((BLK, 8), lambda i: (i, 0)),
            pl.BlockSpec((1, hid), lambda i: (0, 0)),
            pl.BlockSpec((emb, hid), lambda i: (0, 0)),
        ],
        out_specs=pl.BlockSpec((BLK, emb), lambda i: (i, 0)),
        out_shape=jax.ShapeDtypeStruct((N_PAD, emb), jnp.float32),
    )(agg1, hs1, dinv8, b1.reshape(1, hid), W2)

    agg2 = _make_edge_kernel(cpw, emb)(srcp, dstp, hs2, zeros_e)

    out = pl.pallas_call(
        _final_body,
        grid=grid,
        in_specs=[
            pl.BlockSpec((2, BLK, emb), lambda i: (0, i, 0)),
            pl.BlockSpec((BLK, emb), lambda i: (i, 0)),
            pl.BlockSpec((BLK, 8), lambda i: (i, 0)),
            pl.BlockSpec((1, emb), lambda i: (0, 0)),
        ],
        out_specs=pl.BlockSpec((BLK, emb), lambda i: (i, 0)),
        out_shape=jax.ShapeDtypeStruct((N_PAD, emb), jnp.float32),
    )(agg2, hs2, dinv8, b2.reshape(1, emb))

    return out[:n]


# speedup vs baseline: 32.4908x; 1.1277x over previous
import functools

import jax
import jax.numpy as jnp
from jax import lax
from jax.experimental import pallas as pl
from jax.experimental.pallas import tpu as pltpu
from jax.experimental.pallas import tpu_sc as plsc

N_PAD = 10240
CHUNK = 128
NWORK = 32
BLK = 1024


def _enc_body(x_ref, wl_ref, bl_ref, w1_ref, o_ref):
    h0 = jnp.dot(x_ref[...], wl_ref[...].T, preferred_element_type=jnp.float32)
    h0 = jnp.maximum(h0 + bl_ref[...], 0.0)
    o_ref[...] = jnp.dot(h0, w1_ref[...].T, preferred_element_type=jnp.float32)


def _scale_body(h1_ref, degp_ref, hs_ref, dinv_ref):
    deg = degp_ref[0] + degp_ref[1] + 1.0
    dinv = lax.rsqrt(deg)
    hs_ref[...] = h1_ref[...] * dinv[:, :1]
    dinv_ref[...] = dinv


def _layer1_body(agg_ref, hs_ref, dinv_ref, b1_ref, w2_ref, o_ref):
    dinv = dinv_ref[:, :1]
    z = (agg_ref[0] + agg_ref[1] + hs_ref[...]) * dinv + b1_ref[...]
    z = jnp.maximum(z, 0.0)
    h2 = jnp.dot(z, w2_ref[...].T, preferred_element_type=jnp.float32)
    o_ref[...] = h2 * dinv


def _final_body(agg_ref, hs2_ref, dinv_ref, b2_ref, o_ref):
    dinv = dinv_ref[:, :1]
    o_ref[...] = (agg_ref[0] + agg_ref[1] + hs2_ref[...]) * dinv + b2_ref[...]


_SC_PARAMS = pltpu.CompilerParams(use_tc_tiling_on_sc=False)


def _make_deg_kernel(cpw):
    mesh = plsc.VectorSubcoreMesh(core_axis_name="c", subcore_axis_name="s")

    @functools.partial(
        pl.kernel,
        out_type=jax.ShapeDtypeStruct((2, N_PAD, 8), jnp.float32),
        mesh=mesh,
        compiler_params=_SC_PARAMS,
        scratch_types=[
            pltpu.VMEM((cpw, CHUNK), jnp.int32),
            pltpu.VMEM((CHUNK, 8), jnp.float32),
            pltpu.VMEM_SHARED((N_PAD, 8), jnp.float32),
        ],
    )
    def deg_kernel(dst_hbm, ones_hbm, zeros_hbm, out_hbm, idx_v, ones_v, acc_s):
        c = lax.axis_index("c")
        s = lax.axis_index("s")
        wid = s * 2 + c
        rpt = N_PAD // 16
        r0 = s * rpt
        pltpu.sync_copy(zeros_hbm.at[pl.ds(r0, rpt)], acc_s.at[pl.ds(r0, rpt)])
        pltpu.sync_copy(ones_hbm, ones_v)
        pltpu.sync_copy(dst_hbm.at[wid], idx_v)
        plsc.subcore_barrier()

        def body(j, carry):
            pltpu.sync_copy(ones_v, acc_s.at[idx_v.at[j]], add=True)
            return carry

        lax.fori_loop(0, cpw, body, 0)
        plsc.subcore_barrier()
        pltpu.sync_copy(acc_s.at[pl.ds(r0, rpt)], out_hbm.at[c, pl.ds(r0, rpt)])

    return deg_kernel


def _make_edge_kernel(cpw, d):
    mesh = plsc.VectorSubcoreMesh(core_axis_name="c", subcore_axis_name="s")

    @functools.partial(
        pl.kernel,
        out_type=jax.ShapeDtypeStruct((2, N_PAD, d), jnp.float32),
        mesh=mesh,
        compiler_params=_SC_PARAMS,
        scratch_types=[
            pltpu.VMEM((cpw, CHUNK), jnp.int32),
            pltpu.VMEM((cpw, CHUNK), jnp.int32),
            pltpu.VMEM((CHUNK, d), jnp.float32),
            pltpu.VMEM((CHUNK, d), jnp.float32),
            pltpu.VMEM_SHARED((N_PAD, d), jnp.float32),
            pltpu.SemaphoreType.DMA,
            pltpu.SemaphoreType.DMA,
        ],
    )
    def edge_kernel(src_hbm, dst_hbm, table_hbm, zeros_hbm, out_hbm,
                    src_v, dst_v, rows0_v, rows1_v, acc_s, sem0, sem1):
        c = lax.axis_index("c")
        s = lax.axis_index("s")
        wid = s * 2 + c
        rpt = N_PAD // 16
        r0 = s * rpt
        half = cpw // 2
        pltpu.sync_copy(src_hbm.at[wid], src_v)
        pltpu.sync_copy(dst_hbm.at[wid], dst_v)
        pltpu.async_copy(table_hbm.at[src_v.at[0]], rows0_v, sem0)
        pltpu.async_copy(table_hbm.at[src_v.at[1]], rows1_v, sem1)
        pltpu.sync_copy(zeros_hbm.at[pl.ds(r0, rpt)], acc_s.at[pl.ds(r0, rpt)])
        plsc.subcore_barrier()

        def body(t, carry):
            j0 = 2 * t
            pltpu.make_async_copy(table_hbm.at[src_v.at[j0]], rows0_v, sem0).wait()
            pltpu.sync_copy(rows0_v, acc_s.at[dst_v.at[j0]], add=True)

            @pl.when(t < half - 1)
            def _():
                pltpu.async_copy(table_hbm.at[src_v.at[j0 + 2]], rows0_v, sem0)

            pltpu.make_async_copy(table_hbm.at[src_v.at[j0 + 1]], rows1_v, sem1).wait()
            pltpu.sync_copy(rows1_v, acc_s.at[dst_v.at[j0 + 1]], add=True)

            @pl.when(t < half - 1)
            def _():
                pltpu.async_copy(table_hbm.at[src_v.at[j0 + 3]], rows1_v, sem1)

            return carry

        lax.fori_loop(0, half, body, 0)
        plsc.subcore_barrier()
        pltpu.sync_copy(acc_s.at[pl.ds(r0, rpt)], out_hbm.at[c, pl.ds(r0, rpt)])

    return edge_kernel


def kernel(x, ei, W_lin, b_lin, W1, b1, W2, b2):
    n, feat = x.shape
    hid = W1.shape[0]
    emb = W2.shape[0]
    e = ei.shape[1]
    cpw = -(-e // (NWORK * CHUNK))
    cpw = cpw + (cpw % 2)
    ep = NWORK * cpw * CHUNK

    xp = jnp.zeros((N_PAD, feat), jnp.float32).at[:n].set(x)
    pad = jnp.full((ep - e,), n, dtype=jnp.int32)
    srcp = jnp.concatenate([ei[0], pad]).reshape(NWORK, cpw, CHUNK)
    dstp = jnp.concatenate([ei[1], pad]).reshape(NWORK, cpw, CHUNK)
    ones8 = jnp.ones((CHUNK, 8), jnp.float32)
    zeros8 = jnp.zeros((N_PAD, 8), jnp.float32)
    zeros_h = jnp.zeros((N_PAD, hid), jnp.float32)
    zeros_e = jnp.zeros((N_PAD, emb), jnp.float32)

    grid = (N_PAD // BLK,)

    h1 = pl.pallas_call(
        _enc_body,
        grid=grid,
        in_specs=[
            pl.BlockSpec((BLK, feat), lambda i: (i, 0)),
            pl.BlockSpec((hid, feat), lambda i: (0, 0)),
            pl.BlockSpec((1, hid), lambda i: (0, 0)),
            pl.BlockSpec((hid, hid), lambda i: (0, 0)),
        ],
        out_specs=pl.BlockSpec((BLK, hid), lambda i: (i, 0)),
        out_shape=jax.ShapeDtypeStruct((N_PAD, hid), jnp.float32),
    )(xp, W_lin, b_lin.reshape(1, hid), W1)

    degp = _make_deg_kernel(cpw)(dstp, ones8, zeros8)

    hs1, dinv8 = pl.pallas_call(
        _scale_body,
        grid=grid,
        in_specs=[
            pl.BlockSpec((BLK, hid), lambda i: (i, 0)),
            pl.BlockSpec((2, BLK, 8), lambda i: (0, i, 0)),
        ],
        out_specs=[
            pl.BlockSpec((BLK, hid), lambda i: (i, 0)),
            pl.BlockSpec((BLK, 8), lambda i: (i, 0)),
        ],
        out_shape=[
            jax.ShapeDtypeStruct((N_PAD, hid), jnp.float32),
            jax.ShapeDtypeStruct((N_PAD, 8), jnp.float32),
        ],
    )(h1, degp)

    agg1 = _make_edge_kernel(cpw, hid)(srcp, dstp, hs1, zeros_h)

    hs2 = pl.pallas_call(
        _layer1_body,
        grid=grid,
        in_specs=[
            pl.BlockSpec((2, BLK, hid), lambda i: (0, i, 0)),
            pl.BlockSpec((BLK, hid), lambda i: (i, 0)),
            pl.BlockSpec((BLK, 8), lambda i: (i, 0)),
            pl.BlockSpec((1, hid), lambda i: (0, 0)),
            pl.BlockSpec((emb, hid), lambda i: (0, 0)),
        ],
        out_specs=pl.BlockSpec((BLK, emb), lambda i: (i, 0)),
        out_shape=jax.ShapeDtypeStruct((N_PAD, emb), jnp.float32),
    )(agg1, hs1, dinv8, b1.reshape(1, hid), W2)

    agg2 = _make_edge_kernel(cpw, emb)(srcp, dstp, hs2, zeros_e)

    out = pl.pallas_call(
        _final_body,
        grid=grid,
        in_specs=[
            pl.BlockSpec((2, BLK, emb), lambda i: (0, i, 0)),
            pl.BlockSpec((BLK, emb), lambda i: (i, 0)),
            pl.BlockSpec((BLK, 8), lambda i: (i, 0)),
            pl.BlockSpec((1, emb), lambda i: (0, 0)),
        ],
        out_specs=pl.BlockSpec((BLK, emb), lambda i: (i, 0)),
        out_shape=jax.ShapeDtypeStruct((N_PAD, emb), jnp.float32),
    )(agg2, hs2, dinv8, b2.reshape(1, emb))

    return out[:n]


# speedup vs baseline: 45.7326x; 1.4076x over previous
import functools

import jax
import jax.numpy as jnp
from jax import lax
from jax.experimental import pallas as pl
from jax.experimental.pallas import tpu as pltpu
from jax.experimental.pallas import tpu_sc as plsc

CHUNK = 125
NWORK = 32


def _enc_body(x_ref, wl_ref, bl_ref, w1_ref, o_ref):
    h0 = jnp.dot(x_ref[...], wl_ref[...].T, preferred_element_type=jnp.float32)
    h0 = jnp.maximum(h0 + bl_ref[...], 0.0)
    o_ref[...] = jnp.dot(h0, w1_ref[...].T, preferred_element_type=jnp.float32)


def _scale_body(h1_ref, degp_ref, hs_ref, dinv_ref):
    deg = degp_ref[0] + degp_ref[1] + 1.0
    dinv = lax.rsqrt(deg)
    hs_ref[...] = h1_ref[...] * dinv[:, :1]
    dinv_ref[...] = dinv


def _layer1_body(agg_ref, hs_ref, dinv_ref, b1_ref, w2_ref, o_ref):
    dinv = dinv_ref[:, :1]
    z = (agg_ref[0] + agg_ref[1] + hs_ref[...]) * dinv + b1_ref[...]
    z = jnp.maximum(z, 0.0)
    h2 = jnp.dot(z, w2_ref[...].T, preferred_element_type=jnp.float32)
    o_ref[...] = h2 * dinv


def _final_body(agg_ref, hs2_ref, dinv_ref, b2_ref, o_ref):
    dinv = dinv_ref[:, :1]
    o_ref[...] = (agg_ref[0] + agg_ref[1] + hs2_ref[...]) * dinv + b2_ref[...]


_SC_PARAMS = pltpu.CompilerParams(use_tc_tiling_on_sc=False)


def _make_deg_kernel(n, cpw):
    mesh = plsc.VectorSubcoreMesh(core_axis_name="c", subcore_axis_name="s")

    @functools.partial(
        pl.kernel,
        out_type=jax.ShapeDtypeStruct((2, n, 8), jnp.float32),
        mesh=mesh,
        compiler_params=_SC_PARAMS,
        scratch_types=[
            pltpu.VMEM((cpw, CHUNK), jnp.int32),
            pltpu.VMEM((CHUNK, 8), jnp.float32),
            pltpu.VMEM_SHARED((n, 8), jnp.float32),
        ],
    )
    def deg_kernel(dst_hbm, ones_hbm, zeros_hbm, out_hbm, idx_v, ones_v, acc_s):
        c = lax.axis_index("c")
        s = lax.axis_index("s")
        wid = s * 2 + c
        rpt = n // 16
        r0 = s * rpt
        pltpu.sync_copy(zeros_hbm.at[pl.ds(r0, rpt)], acc_s.at[pl.ds(r0, rpt)])
        pltpu.sync_copy(ones_hbm, ones_v)
        pltpu.sync_copy(dst_hbm.at[wid], idx_v)
        plsc.subcore_barrier()

        def body(j, carry):
            pltpu.sync_copy(ones_v, acc_s.at[idx_v.at[j]], add=True)
            return carry

        lax.fori_loop(0, cpw, body, 0)
        plsc.subcore_barrier()
        pltpu.sync_copy(acc_s.at[pl.ds(r0, rpt)], out_hbm.at[c, pl.ds(r0, rpt)])

    return deg_kernel


def _make_edge_kernel(n, cpw, d):
    mesh = plsc.VectorSubcoreMesh(core_axis_name="c", subcore_axis_name="s")

    @functools.partial(
        pl.kernel,
        out_type=jax.ShapeDtypeStruct((2, n, d), jnp.float32),
        mesh=mesh,
        compiler_params=_SC_PARAMS,
        scratch_types=[
            pltpu.VMEM((cpw, CHUNK), jnp.int32),
            pltpu.VMEM((cpw, CHUNK), jnp.int32),
            pltpu.VMEM((CHUNK, d), jnp.float32),
            pltpu.VMEM((CHUNK, d), jnp.float32),
            pltpu.VMEM_SHARED((n, d), jnp.float32),
            pltpu.SemaphoreType.DMA,
            pltpu.SemaphoreType.DMA,
        ],
    )
    def edge_kernel(src_hbm, dst_hbm, table_hbm, zeros_hbm, out_hbm,
                    src_v, dst_v, rows0_v, rows1_v, acc_s, sem0, sem1):
        c = lax.axis_index("c")
        s = lax.axis_index("s")
        wid = s * 2 + c
        rpt = n // 16
        r0 = s * rpt
        half = cpw // 2
        pltpu.sync_copy(src_hbm.at[wid], src_v)
        pltpu.sync_copy(dst_hbm.at[wid], dst_v)
        pltpu.async_copy(table_hbm.at[src_v.at[0]], rows0_v, sem0)
        pltpu.async_copy(table_hbm.at[src_v.at[1]], rows1_v, sem1)
        pltpu.sync_copy(zeros_hbm.at[pl.ds(r0, rpt)], acc_s.at[pl.ds(r0, rpt)])
        plsc.subcore_barrier()

        def body(t, carry):
            j0 = 2 * t
            pltpu.make_async_copy(table_hbm.at[src_v.at[j0]], rows0_v, sem0).wait()
            pltpu.sync_copy(rows0_v, acc_s.at[dst_v.at[j0]], add=True)

            @pl.when(t < half - 1)
            def _():
                pltpu.async_copy(table_hbm.at[src_v.at[j0 + 2]], rows0_v, sem0)

            pltpu.make_async_copy(table_hbm.at[src_v.at[j0 + 1]], rows1_v, sem1).wait()
            pltpu.sync_copy(rows1_v, acc_s.at[dst_v.at[j0 + 1]], add=True)

            @pl.when(t < half - 1)
            def _():
                pltpu.async_copy(table_hbm.at[src_v.at[j0 + 3]], rows1_v, sem1)

            return carry

        lax.fori_loop(0, half, body, 0)
        plsc.subcore_barrier()
        pltpu.sync_copy(acc_s.at[pl.ds(r0, rpt)], out_hbm.at[c, pl.ds(r0, rpt)])

    return edge_kernel


def kernel(x, ei, W_lin, b_lin, W1, b1, W2, b2):
    n, feat = x.shape
    hid = W1.shape[0]
    emb = W2.shape[0]
    e = ei.shape[1]
    cpw = e // (NWORK * CHUNK)

    srcp = ei[0].reshape(NWORK, cpw, CHUNK)
    dstp = ei[1].reshape(NWORK, cpw, CHUNK)
    ones8 = jnp.ones((CHUNK, 8), jnp.float32)
    zeros8 = jnp.zeros((n, 8), jnp.float32)
    zeros_h = jnp.zeros((n, hid), jnp.float32)
    zeros_e = jnp.zeros((n, emb), jnp.float32)

    h1 = pl.pallas_call(
        _enc_body,
        out_shape=jax.ShapeDtypeStruct((n, hid), jnp.float32),
    )(x, W_lin, b_lin.reshape(1, hid), W1)

    degp = _make_deg_kernel(n, cpw)(dstp, ones8, zeros8)

    hs1, dinv8 = pl.pallas_call(
        _scale_body,
        out_shape=[
            jax.ShapeDtypeStruct((n, hid), jnp.float32),
            jax.ShapeDtypeStruct((n, 8), jnp.float32),
        ],
    )(h1, degp)

    agg1 = _make_edge_kernel(n, cpw, hid)(srcp, dstp, hs1, zeros_h)

    hs2 = pl.pallas_call(
        _layer1_body,
        out_shape=jax.ShapeDtypeStruct((n, emb), jnp.float32),
    )(agg1, hs1, dinv8, b1.reshape(1, hid), W2)

    agg2 = _make_edge_kernel(n, cpw, emb)(srcp, dstp, hs2, zeros_e)

    out = pl.pallas_call(
        _final_body,
        out_shape=jax.ShapeDtypeStruct((n, emb), jnp.float32),
    )(agg2, hs2, dinv8, b2.reshape(1, emb))

    return out


# speedup vs baseline: 56.2502x; 1.2300x over previous
import functools

import jax
import jax.numpy as jnp
from jax import lax
from jax.experimental import pallas as pl
from jax.experimental.pallas import tpu as pltpu
from jax.experimental.pallas import tpu_sc as plsc

CHUNK = 125
NWORK = 32


def _enc_body(x_ref, wl_ref, bl_ref, w1_ref, o_ref):
    h0 = jnp.dot(x_ref[...], wl_ref[...].T, preferred_element_type=jnp.float32)
    h0 = jnp.maximum(h0 + bl_ref[...], 0.0)
    o_ref[...] = jnp.dot(h0, w1_ref[...].T, preferred_element_type=jnp.float32)


def _scale_body(h1_ref, degp_ref, hs_ref, dinv_ref):
    deg = degp_ref[0] + degp_ref[1] + 1.0
    dinv = lax.rsqrt(deg)
    hs_ref[...] = h1_ref[...] * dinv[:, :1]
    dinv_ref[...] = dinv


def _layer1_body(agg_ref, hs_ref, dinv_ref, b1_ref, w2_ref, o_ref):
    dinv = dinv_ref[:, :1]
    z = (agg_ref[0] + agg_ref[1] + hs_ref[...]) * dinv + b1_ref[...]
    z = jnp.maximum(z, 0.0)
    h2 = jnp.dot(z, w2_ref[...].T, preferred_element_type=jnp.float32)
    o_ref[...] = h2 * dinv


def _final_body(agg_ref, hs2_ref, dinv_ref, b2_ref, o_ref):
    dinv = dinv_ref[:, :1]
    o_ref[...] = (agg_ref[0] + agg_ref[1] + hs2_ref[...]) * dinv + b2_ref[...]


_SC_PARAMS = pltpu.CompilerParams(use_tc_tiling_on_sc=False)


def _make_deg_kernel(n, cpw):
    mesh = plsc.VectorSubcoreMesh(core_axis_name="c", subcore_axis_name="s")

    @functools.partial(
        pl.kernel,
        out_type=jax.ShapeDtypeStruct((2, n, 8), jnp.float32),
        mesh=mesh,
        compiler_params=_SC_PARAMS,
        scratch_types=[
            pltpu.VMEM((cpw, CHUNK), jnp.int32),
            pltpu.VMEM((CHUNK, 8), jnp.float32),
            pltpu.VMEM_SHARED((n, 8), jnp.float32),
            pltpu.SemaphoreType.DMA,
            pltpu.SemaphoreType.DMA,
            pltpu.SemaphoreType.DMA,
            pltpu.SemaphoreType.DMA,
        ],
    )
    def deg_kernel(ei_hbm, ones_hbm, zeros_hbm, out_hbm, idx_v, ones_v, acc_s,
                   s0, s1, s2, s3):
        c = lax.axis_index("c")
        s = lax.axis_index("s")
        wid = s * 2 + c
        rpt = n // 16
        r0 = s * rpt
        ssems = (s0, s1, s2, s3)
        pltpu.sync_copy(zeros_hbm.at[pl.ds(r0, rpt)], acc_s.at[pl.ds(r0, rpt)])
        pltpu.sync_copy(ones_hbm, ones_v)
        pltpu.sync_copy(ei_hbm.at[1, wid], idx_v)
        plsc.subcore_barrier()

        def body(t, carry):
            for b in range(4):
                j = 4 * t + b

                @pl.when(t > 0)
                def _():
                    pltpu.make_async_copy(
                        ones_v, acc_s.at[idx_v.at[j - 4]], ssems[b]).wait()

                pltpu.async_copy(ones_v, acc_s.at[idx_v.at[j]], ssems[b],
                                 add=True)
            return carry

        lax.fori_loop(0, cpw // 4, body, 0)
        for b in range(4):
            pltpu.make_async_copy(
                ones_v, acc_s.at[idx_v.at[cpw - 4 + b]], ssems[b]).wait()
        plsc.subcore_barrier()
        pltpu.sync_copy(acc_s.at[pl.ds(r0, rpt)], out_hbm.at[c, pl.ds(r0, rpt)])

    return deg_kernel


def _make_edge_kernel(n, cpw, d):
    mesh = plsc.VectorSubcoreMesh(core_axis_name="c", subcore_axis_name="s")

    @functools.partial(
        pl.kernel,
        out_type=jax.ShapeDtypeStruct((2, n, d), jnp.float32),
        mesh=mesh,
        compiler_params=_SC_PARAMS,
        scratch_types=[
            pltpu.VMEM((cpw, CHUNK), jnp.int32),
            pltpu.VMEM((cpw, CHUNK), jnp.int32),
            pltpu.VMEM((CHUNK, d), jnp.float32),
            pltpu.VMEM((CHUNK, d), jnp.float32),
            pltpu.VMEM((CHUNK, d), jnp.float32),
            pltpu.VMEM((CHUNK, d), jnp.float32),
            pltpu.VMEM_SHARED((n, d), jnp.float32),
            pltpu.SemaphoreType.DMA,
            pltpu.SemaphoreType.DMA,
            pltpu.SemaphoreType.DMA,
            pltpu.SemaphoreType.DMA,
            pltpu.SemaphoreType.DMA,
            pltpu.SemaphoreType.DMA,
            pltpu.SemaphoreType.DMA,
            pltpu.SemaphoreType.DMA,
        ],
    )
    def edge_kernel(ei_hbm, table_hbm, zeros_hbm, out_hbm,
                    src_v, dst_v, r0_v, r1_v, r2_v, r3_v, acc_s,
                    g0, g1, g2, g3, t0, t1, t2, t3):
        c = lax.axis_index("c")
        s = lax.axis_index("s")
        wid = s * 2 + c
        rpt = n // 16
        rbase = s * rpt
        rows = (r0_v, r1_v, r2_v, r3_v)
        gsems = (g0, g1, g2, g3)
        ssems = (t0, t1, t2, t3)
        pltpu.sync_copy(ei_hbm.at[0, wid], src_v)
        pltpu.sync_copy(ei_hbm.at[1, wid], dst_v)
        for b in range(4):
            pltpu.async_copy(table_hbm.at[src_v.at[b]], rows[b], gsems[b])
        pltpu.sync_copy(zeros_hbm.at[pl.ds(rbase, rpt)],
                        acc_s.at[pl.ds(rbase, rpt)])
        plsc.subcore_barrier()

        def body(t, carry):
            for b in range(4):
                j = 4 * t + b
                pltpu.make_async_copy(
                    table_hbm.at[src_v.at[j]], rows[b], gsems[b]).wait()
                pltpu.async_copy(rows[b], acc_s.at[dst_v.at[j]], ssems[b],
                                 add=True)
            for b in range(4):
                j = 4 * t + b
                pltpu.make_async_copy(
                    rows[b], acc_s.at[dst_v.at[j]], ssems[b]).wait()

                @pl.when(j + 4 < cpw)
                def _():
                    pltpu.async_copy(
                        table_hbm.at[src_v.at[j + 4]], rows[b], gsems[b])
            return carry

        lax.fori_loop(0, cpw // 4, body, 0)
        plsc.subcore_barrier()
        pltpu.sync_copy(acc_s.at[pl.ds(rbase, rpt)],
                        out_hbm.at[c, pl.ds(rbase, rpt)])

    return edge_kernel


def kernel(x, ei, W_lin, b_lin, W1, b1, W2, b2):
    n, feat = x.shape
    hid = W1.shape[0]
    emb = W2.shape[0]
    e = ei.shape[1]
    cpw = e // (NWORK * CHUNK)

    ei_r = ei.reshape(2, NWORK, cpw, CHUNK)
    ones8 = jnp.ones((CHUNK, 8), jnp.float32)
    zeros8 = jnp.zeros((n, 8), jnp.float32)
    zeros_h = jnp.zeros((n, hid), jnp.float32)
    zeros_e = jnp.zeros((n, emb), jnp.float32)

    h1 = pl.pallas_call(
        _enc_body,
        out_shape=jax.ShapeDtypeStruct((n, hid), jnp.float32),
    )(x, W_lin, b_lin.reshape(1, hid), W1)

    degp = _make_deg_kernel(n, cpw)(ei_r, ones8, zeros8)

    hs1, dinv8 = pl.pallas_call(
        _scale_body,
        out_shape=[
            jax.ShapeDtypeStruct((n, hid), jnp.float32),
            jax.ShapeDtypeStruct((n, 8), jnp.float32),
        ],
    )(h1, degp)

    agg1 = _make_edge_kernel(n, cpw, hid)(ei_r, hs1, zeros_h)

    hs2 = pl.pallas_call(
        _layer1_body,
        out_shape=jax.ShapeDtypeStruct((n, emb), jnp.float32),
    )(agg1, hs1, dinv8, b1.reshape(1, hid), W2)

    agg2 = _make_edge_kernel(n, cpw, emb)(ei_r, hs2, zeros_e)

    out = pl.pallas_call(
        _final_body,
        out_shape=jax.ShapeDtypeStruct((n, emb), jnp.float32),
    )(agg2, hs2, dinv8, b2.reshape(1, emb))

    return out


# speedup vs baseline: 71.1133x; 1.2642x over previous
import functools

import jax
import jax.numpy as jnp
from jax import lax
from jax.experimental import pallas as pl
from jax.experimental.pallas import tpu as pltpu
from jax.experimental.pallas import tpu_sc as plsc

CHUNK = 125
NWORK = 32


def _enc_body(x_ref, wl_ref, bl_ref, w1_ref, o_ref, *, n_pad):
    h0 = jnp.dot(x_ref[...], wl_ref[...].T, preferred_element_type=jnp.float32)
    h0 = jnp.maximum(h0 + bl_ref[...], 0.0)
    h1 = jnp.dot(h0, w1_ref[...].T, preferred_element_type=jnp.float32)
    pad = jnp.zeros((n_pad - h1.shape[0], h1.shape[1]), jnp.float32)
    o_ref[...] = jnp.concatenate([h1, pad], axis=0)


def _scale_body(h1p_ref, degp_ref, hs_ref, dinv_ref):
    dinv = lax.rsqrt(degp_ref[0] + degp_ref[1] + 1.0)
    hs_ref[...] = h1p_ref[...] * dinv
    dinv_ref[...] = dinv


def _layer1_body(aggp_ref, hsp_ref, dinvp_ref, b1t_ref, w2_ref, o_ref,
                 *, hid):
    dinv = dinvp_ref[...]
    z = (aggp_ref[0] + aggp_ref[1] + hsp_ref[...]) * dinv + b1t_ref[...]
    z = jnp.maximum(z, 0.0)
    w2 = w2_ref[...]
    w2pad = jnp.concatenate(
        [w2, jnp.zeros((w2.shape[1] - w2.shape[0], w2.shape[1]),
                       jnp.float32)], axis=0)
    parts = [
        jnp.dot(z[:, j * hid:(j + 1) * hid], w2pad.T,
                preferred_element_type=jnp.float32)
        for j in range(128 // hid)
    ]
    o_ref[...] = jnp.concatenate(parts, axis=1) * dinv


def _final_body(aggp_ref, hs2p_ref, dinvp_ref, b2t_ref, o_ref):
    o_ref[...] = (aggp_ref[0] + aggp_ref[1] + hs2p_ref[...]) \
        * dinvp_ref[...] + b2t_ref[...]


_SC_PARAMS = pltpu.CompilerParams(use_tc_tiling_on_sc=False,
                                  needs_layout_passes=False)


def _make_deg_kernel(n, cpw):
    mesh = plsc.VectorSubcoreMesh(core_axis_name="c", subcore_axis_name="s")

    @functools.partial(
        pl.kernel,
        out_type=jax.ShapeDtypeStruct((2, n, 32), jnp.float32),
        mesh=mesh,
        compiler_params=_SC_PARAMS,
        scratch_types=[
            pltpu.VMEM((cpw, CHUNK), jnp.int32),
            pltpu.VMEM((CHUNK, 8), jnp.float32),
            pltpu.VMEM((n // 16, 8), jnp.float32),
            pltpu.VMEM((n // 16, 32), jnp.float32),
            pltpu.VMEM_SHARED((n, 8), jnp.float32),
            pltpu.SemaphoreType.DMA,
            pltpu.SemaphoreType.DMA,
            pltpu.SemaphoreType.DMA,
            pltpu.SemaphoreType.DMA,
        ],
    )
    def deg_kernel(ei_hbm, ones_hbm, zeros_hbm, out_hbm, idx_v, ones_v,
                   deg8_v, deg32_v, acc_s, s0, s1, s2, s3):
        c = lax.axis_index("c")
        s = lax.axis_index("s")
        wid = s * 2 + c
        rpt = n // 16
        r0 = s * rpt
        ssems = (s0, s1, s2, s3)
        pltpu.sync_copy(zeros_hbm.at[pl.ds(r0, rpt)], acc_s.at[pl.ds(r0, rpt)])
        pltpu.sync_copy(ones_hbm, ones_v)
        pltpu.sync_copy(ei_hbm.at[1, wid], idx_v)
        plsc.subcore_barrier()

        def body(t, carry):
            for b in range(4):
                j = 4 * t + b

                @pl.when(t > 0)
                def _():
                    pltpu.make_async_copy(
                        ones_v, acc_s.at[idx_v.at[j - 4]], ssems[b]).wait()

                pltpu.async_copy(ones_v, acc_s.at[idx_v.at[j]], ssems[b],
                                 add=True)
            return carry

        lax.fori_loop(0, cpw // 4, body, 0)
        for b in range(4):
            pltpu.make_async_copy(
                ones_v, acc_s.at[idx_v.at[cpw - 4 + b]], ssems[b]).wait()
        plsc.subcore_barrier()
        pltpu.sync_copy(acc_s.at[pl.ds(r0, rpt)], deg8_v)
        zcol = jnp.zeros((16,), jnp.int32)

        def expand(r, carry):
            dv = plsc.load_gather(deg8_v, [jnp.full((16,), r, jnp.int32),
                                           zcol])
            deg32_v[r, pl.ds(0, 16)] = dv
            deg32_v[r, pl.ds(16, 16)] = dv
            return carry

        lax.fori_loop(0, rpt, expand, 0)
        pltpu.sync_copy(deg32_v, out_hbm.at[c, pl.ds(r0, rpt)])

    return deg_kernel


def _make_edge_kernel(n, cpw, d):
    mesh = plsc.VectorSubcoreMesh(core_axis_name="c", subcore_axis_name="s")

    @functools.partial(
        pl.kernel,
        out_type=jax.ShapeDtypeStruct((2, n, d), jnp.float32),
        mesh=mesh,
        compiler_params=_SC_PARAMS,
        scratch_types=[
            pltpu.VMEM((cpw, CHUNK), jnp.int32),
            pltpu.VMEM((cpw, CHUNK), jnp.int32),
            pltpu.VMEM((CHUNK, d), jnp.float32),
            pltpu.VMEM((CHUNK, d), jnp.float32),
            pltpu.VMEM((CHUNK, d), jnp.float32),
            pltpu.VMEM((CHUNK, d), jnp.float32),
            pltpu.VMEM_SHARED((n, d), jnp.float32),
            pltpu.SemaphoreType.DMA,
            pltpu.SemaphoreType.DMA,
            pltpu.SemaphoreType.DMA,
            pltpu.SemaphoreType.DMA,
            pltpu.SemaphoreType.DMA,
            pltpu.SemaphoreType.DMA,
            pltpu.SemaphoreType.DMA,
            pltpu.SemaphoreType.DMA,
        ],
    )
    def edge_kernel(ei_hbm, table_hbm, zeros_hbm, out_hbm,
                    src_v, dst_v, r0_v, r1_v, r2_v, r3_v, acc_s,
                    g0, g1, g2, g3, t0, t1, t2, t3):
        c = lax.axis_index("c")
        s = lax.axis_index("s")
        wid = s * 2 + c
        rpt = n // 16
        rbase = s * rpt
        rows = (r0_v, r1_v, r2_v, r3_v)
        gsems = (g0, g1, g2, g3)
        ssems = (t0, t1, t2, t3)
        pltpu.sync_copy(ei_hbm.at[0, wid], src_v)
        pltpu.sync_copy(ei_hbm.at[1, wid], dst_v)
        for b in range(4):
            pltpu.async_copy(table_hbm.at[src_v.at[b]], rows[b], gsems[b])
        pltpu.sync_copy(zeros_hbm.at[pl.ds(rbase, rpt)],
                        acc_s.at[pl.ds(rbase, rpt)])
        plsc.subcore_barrier()

        def body(t, carry):
            for b in range(4):
                j = 4 * t + b
                pltpu.make_async_copy(
                    table_hbm.at[src_v.at[j]], rows[b], gsems[b]).wait()
                pltpu.async_copy(rows[b], acc_s.at[dst_v.at[j]], ssems[b],
                                 add=True)
            for b in range(4):
                j = 4 * t + b
                pltpu.make_async_copy(
                    rows[b], acc_s.at[dst_v.at[j]], ssems[b]).wait()

                @pl.when(j + 4 < cpw)
                def _():
                    pltpu.async_copy(
                        table_hbm.at[src_v.at[j + 4]], rows[b], gsems[b])
            return carry

        lax.fori_loop(0, cpw // 4, body, 0)
        plsc.subcore_barrier()
        pltpu.sync_copy(acc_s.at[pl.ds(rbase, rpt)],
                        out_hbm.at[c, pl.ds(rbase, rpt)])

    return edge_kernel


def kernel(x, ei, W_lin, b_lin, W1, b1, W2, b2):
    n, feat = x.shape
    hid = W1.shape[0]
    emb = W2.shape[0]
    e = ei.shape[1]
    cpw = e // (NWORK * CHUNK)
    n_pad = -(-n // 512) * 512
    rh = n_pad * hid // 128

    ei_r = ei.reshape(2, NWORK, cpw, CHUNK)
    ones8 = jnp.ones((CHUNK, 8), jnp.float32)
    zeros8 = jnp.zeros((n_pad, 8), jnp.float32)
    zeros_h = jnp.zeros((n_pad, hid), jnp.float32)

    h1 = pl.pallas_call(
        functools.partial(_enc_body, n_pad=n_pad),
        out_shape=jax.ShapeDtypeStruct((n_pad, hid), jnp.float32),
    )(x, W_lin, b_lin.reshape(1, hid), W1)

    degp = _make_deg_kernel(n_pad, cpw)(ei_r, ones8, zeros8)

    hs1p, dinvp = pl.pallas_call(
        _scale_body,
        out_shape=[
            jax.ShapeDtypeStruct((rh, 128), jnp.float32),
            jax.ShapeDtypeStruct((rh, 128), jnp.float32),
        ],
    )(h1.reshape(rh, 128), degp.reshape(2, rh, 128))

    edge = _make_edge_kernel(n_pad, cpw, hid)
    agg1 = edge(ei_r, hs1p.reshape(n_pad, hid), zeros_h)

    hs2p = pl.pallas_call(
        functools.partial(_layer1_body, hid=hid),
        out_shape=jax.ShapeDtypeStruct((rh, 128), jnp.float32),
    )(agg1.reshape(2, rh, 128), hs1p, dinvp,
      jnp.tile(b1, 128 // hid).reshape(1, 128), W2)

    agg2 = edge(ei_r, hs2p.reshape(n_pad, hid), zeros_h)

    b2t = jnp.tile(jnp.concatenate([b2, jnp.zeros((hid - emb,), jnp.float32)]),
                   128 // hid).reshape(1, 128)
    outp = pl.pallas_call(
        _final_body,
        out_shape=jax.ShapeDtypeStruct((rh, 128), jnp.float32),
    )(agg2.reshape(2, rh, 128), hs2p, dinvp, b2t)

    return outp.reshape(n_pad, hid)[:n, :emb]


# speedup vs baseline: 72.9543x; 1.0259x over previous
import functools

import jax
import jax.numpy as jnp
from jax import lax
from jax.experimental import pallas as pl
from jax.experimental.pallas import tpu as pltpu
from jax.experimental.pallas import tpu_sc as plsc

CHUNK = 125
NWORK = 32


def _enc_body(x_ref, wl_ref, bl_ref, w1_ref, o_ref, *, n_pad):
    h0 = jnp.dot(x_ref[...], wl_ref[...].T, preferred_element_type=jnp.float32)
    h0 = jnp.maximum(h0 + bl_ref[...], 0.0)
    h1 = jnp.dot(h0, w1_ref[...].T, preferred_element_type=jnp.float32)
    pad = jnp.zeros((n_pad - h1.shape[0], h1.shape[1]), jnp.float32)
    o_ref[...] = jnp.concatenate([h1, pad], axis=0)


def _scale_body(h1p_ref, degp_ref, hs_ref, dinv_ref):
    dinv = lax.rsqrt(degp_ref[0] + degp_ref[1] + 1.0)
    hs_ref[...] = h1p_ref[...] * dinv
    dinv_ref[...] = dinv


def _layer1_body(aggp_ref, hsp_ref, dinvp_ref, b1t_ref, w2_ref, o_ref,
                 *, hid):
    dinv = dinvp_ref[...]
    z = (aggp_ref[0] + aggp_ref[1] + hsp_ref[...]) * dinv + b1t_ref[...]
    z = jnp.maximum(z, 0.0)
    w2 = w2_ref[...]
    w2pad = jnp.concatenate(
        [w2, jnp.zeros((w2.shape[1] - w2.shape[0], w2.shape[1]),
                       jnp.float32)], axis=0)
    parts = [
        jnp.dot(z[:, j * hid:(j + 1) * hid], w2pad.T,
                preferred_element_type=jnp.float32)
        for j in range(128 // hid)
    ]
    o_ref[...] = jnp.concatenate(parts, axis=1) * dinv


def _final_body(aggp_ref, hs2p_ref, dinvp_ref, b2t_ref, o_ref):
    o_ref[...] = (aggp_ref[0] + aggp_ref[1] + hs2p_ref[...]) \
        * dinvp_ref[...] + b2t_ref[...]


_SC_PARAMS = pltpu.CompilerParams(use_tc_tiling_on_sc=False,
                                  needs_layout_passes=False)


def _make_deg_kernel(n, cpw):
    mesh = plsc.VectorSubcoreMesh(core_axis_name="c", subcore_axis_name="s")

    @functools.partial(
        pl.kernel,
        out_type=jax.ShapeDtypeStruct((2, n, 32), jnp.float32),
        mesh=mesh,
        compiler_params=_SC_PARAMS,
        scratch_types=[
            pltpu.VMEM((cpw, CHUNK), jnp.int32),
            pltpu.VMEM((CHUNK, 8), jnp.float32),
            pltpu.VMEM((n // 16, 8), jnp.float32),
            pltpu.VMEM((n // 16, 32), jnp.float32),
            pltpu.VMEM_SHARED((n, 8), jnp.float32),
            pltpu.SemaphoreType.DMA,
            pltpu.SemaphoreType.DMA,
            pltpu.SemaphoreType.DMA,
            pltpu.SemaphoreType.DMA,
        ],
    )
    def deg_kernel(ei_hbm, ones_hbm, zeros_hbm, out_hbm, idx_v, ones_v,
                   deg8_v, deg32_v, acc_s, s0, s1, s2, s3):
        c = lax.axis_index("c")
        s = lax.axis_index("s")
        wid = s * 2 + c
        rpt = n // 16
        r0 = s * rpt
        ssems = (s0, s1, s2, s3)
        pltpu.sync_copy(zeros_hbm.at[pl.ds(r0, rpt)], acc_s.at[pl.ds(r0, rpt)])
        pltpu.sync_copy(ones_hbm, ones_v)
        pltpu.sync_copy(ei_hbm.at[1, wid], idx_v)
        plsc.subcore_barrier()

        def body(t, carry):
            for b in range(4):
                j = 8 * t + 2 * b

                @pl.when(t > 0)
                def _():
                    pltpu.make_async_copy(
                        ones_v, acc_s.at[idx_v.at[j - 8]], ssems[b]).wait()
                    pltpu.make_async_copy(
                        ones_v, acc_s.at[idx_v.at[j - 7]], ssems[b]).wait()

                pltpu.async_copy(ones_v, acc_s.at[idx_v.at[j]], ssems[b],
                                 add=True)
                pltpu.async_copy(ones_v, acc_s.at[idx_v.at[j + 1]], ssems[b],
                                 add=True)
            return carry

        lax.fori_loop(0, cpw // 8, body, 0)
        for b in range(4):
            pltpu.make_async_copy(
                ones_v, acc_s.at[idx_v.at[cpw - 8 + 2 * b]], ssems[b]).wait()
            pltpu.make_async_copy(
                ones_v, acc_s.at[idx_v.at[cpw - 7 + 2 * b]], ssems[b]).wait()
        plsc.subcore_barrier()
        pltpu.sync_copy(acc_s.at[pl.ds(r0, rpt)], deg8_v)
        zcol = jnp.zeros((16,), jnp.int32)
        iota16 = lax.iota(jnp.int32, 16)

        def expand(t, carry):
            ridx = jnp.full((16,), 16 * t, jnp.int32) + iota16
            dv = plsc.load_gather(deg8_v, [ridx, zcol])
            for cc in range(32):
                plsc.store_scatter(
                    deg32_v, [ridx, jnp.full((16,), cc, jnp.int32)], dv)
            return carry

        lax.fori_loop(0, rpt // 16, expand, 0)
        pltpu.sync_copy(deg32_v, out_hbm.at[c, pl.ds(r0, rpt)])

    return deg_kernel


def _make_edge_kernel(n, cpw, d):
    mesh = plsc.VectorSubcoreMesh(core_axis_name="c", subcore_axis_name="s")

    @functools.partial(
        pl.kernel,
        out_type=jax.ShapeDtypeStruct((2, n, d), jnp.float32),
        mesh=mesh,
        compiler_params=_SC_PARAMS,
        scratch_types=(
            [pltpu.VMEM((cpw, CHUNK), jnp.int32),
             pltpu.VMEM((cpw, CHUNK), jnp.int32)]
            + [pltpu.VMEM((CHUNK, d), jnp.float32)] * 8
            + [pltpu.VMEM_SHARED((n, d), jnp.float32)]
            + [pltpu.SemaphoreType.DMA] * 17
        ),
    )
    def edge_kernel(ei_hbm, table_hbm, zeros_hbm, out_hbm,
                    src_v, dst_v, *bufs):
        rows = bufs[0:8]
        acc_s = bufs[8]
        gsems = bufs[9:17]
        ssems = bufs[17:25]
        psem = bufs[25]
        c = lax.axis_index("c")
        s = lax.axis_index("s")
        wid = s * 2 + c
        rpt = n // 16
        rbase = s * rpt
        zslice = zeros_hbm.at[pl.ds(rbase, rpt)]
        aslice = acc_s.at[pl.ds(rbase, rpt)]
        pltpu.async_copy(zslice, aslice, psem)
        pltpu.sync_copy(ei_hbm.at[0, wid], src_v)
        pltpu.sync_copy(ei_hbm.at[1, wid], dst_v)
        for b in range(8):
            pltpu.async_copy(table_hbm.at[src_v.at[b]], rows[b], gsems[b])
        pltpu.make_async_copy(zslice, aslice, psem).wait()
        plsc.subcore_barrier()

        def body(t, carry):
            for b in range(8):
                j = 8 * t + b
                pltpu.make_async_copy(
                    table_hbm.at[src_v.at[j]], rows[b], gsems[b]).wait()
                pltpu.async_copy(rows[b], acc_s.at[dst_v.at[j]], ssems[b],
                                 add=True)
            for b in range(8):
                j = 8 * t + b
                pltpu.make_async_copy(
                    rows[b], acc_s.at[dst_v.at[j]], ssems[b]).wait()

                @pl.when(j + 8 < cpw)
                def _():
                    pltpu.async_copy(
                        table_hbm.at[src_v.at[j + 8]], rows[b], gsems[b])
            return carry

        lax.fori_loop(0, cpw // 8, body, 0)
        plsc.subcore_barrier()
        pltpu.sync_copy(acc_s.at[pl.ds(rbase, rpt)],
                        out_hbm.at[c, pl.ds(rbase, rpt)])

    return edge_kernel


def kernel(x, ei, W_lin, b_lin, W1, b1, W2, b2):
    n, feat = x.shape
    hid = W1.shape[0]
    emb = W2.shape[0]
    e = ei.shape[1]
    cpw = e // (NWORK * CHUNK)
    n_pad = -(-n // 512) * 512
    rh = n_pad * hid // 128

    ei_r = ei.reshape(2, NWORK, cpw, CHUNK)
    ones8 = jnp.ones((CHUNK, 8), jnp.float32)
    zeros8 = jnp.zeros((n_pad, 8), jnp.float32)
    zeros_h = jnp.zeros((n_pad, hid), jnp.float32)

    h1 = pl.pallas_call(
        functools.partial(_enc_body, n_pad=n_pad),
        out_shape=jax.ShapeDtypeStruct((n_pad, hid), jnp.float32),
    )(x, W_lin, b_lin.reshape(1, hid), W1)

    degp = _make_deg_kernel(n_pad, cpw)(ei_r, ones8, zeros8)

    hs1p, dinvp = pl.pallas_call(
        _scale_body,
        out_shape=[
            jax.ShapeDtypeStruct((rh, 128), jnp.float32),
            jax.ShapeDtypeStruct((rh, 128), jnp.float32),
        ],
    )(h1.reshape(rh, 128), degp.reshape(2, rh, 128))

    edge = _make_edge_kernel(n_pad, cpw, hid)
    agg1 = edge(ei_r, hs1p.reshape(n_pad, hid), zeros_h)

    hs2p = pl.pallas_call(
        functools.partial(_layer1_body, hid=hid),
        out_shape=jax.ShapeDtypeStruct((rh, 128), jnp.float32),
    )(agg1.reshape(2, rh, 128), hs1p, dinvp,
      jnp.tile(b1, 128 // hid).reshape(1, 128), W2)

    agg2 = edge(ei_r, hs2p.reshape(n_pad, hid), zeros_h)

    b2t = jnp.tile(jnp.concatenate([b2, jnp.zeros((hid - emb,), jnp.float32)]),
                   128 // hid).reshape(1, 128)
    outp = pl.pallas_call(
        _final_body,
        out_shape=jax.ShapeDtypeStruct((rh, 128), jnp.float32),
    )(agg2.reshape(2, rh, 128), hs2p, dinvp, b2t)

    return outp.reshape(n_pad, hid)[:n, :emb]


# speedup vs baseline: 76.5025x; 1.0486x over previous
import functools

import jax
import jax.numpy as jnp
from jax import lax
from jax.experimental import pallas as pl
from jax.experimental.pallas import tpu as pltpu
from jax.experimental.pallas import tpu_sc as plsc

CHUNK = 125
NWORK = 32


def _enc_body(x_ref, wl_ref, bl_ref, w1_ref, o_ref, *, n_pad):
    h0 = jnp.dot(x_ref[...], wl_ref[...].T, preferred_element_type=jnp.float32)
    h0 = jnp.maximum(h0 + bl_ref[...], 0.0)
    h1 = jnp.dot(h0, w1_ref[...].T, preferred_element_type=jnp.float32)
    pad = jnp.zeros((n_pad - h1.shape[0], h1.shape[1]), jnp.float32)
    o_ref[...] = jnp.concatenate([h1, pad], axis=0)


def _scale_body(h1p_ref, degp_ref, hs_ref, dinv_ref):
    dinv = lax.rsqrt(degp_ref[0] + degp_ref[1] + 1.0)
    hs_ref[...] = h1p_ref[...] * dinv
    dinv_ref[...] = dinv


def _layer1_body(aggp_ref, hsp_ref, dinvp_ref, b1t_ref, w2_ref, o_ref,
                 *, hid):
    dinv = dinvp_ref[...]
    z = (aggp_ref[0] + aggp_ref[1] + hsp_ref[...]) * dinv + b1t_ref[...]
    z = jnp.maximum(z, 0.0)
    w2 = w2_ref[...]
    w2pad = jnp.concatenate(
        [w2, jnp.zeros((w2.shape[1] - w2.shape[0], w2.shape[1]),
                       jnp.float32)], axis=0)
    parts = [
        jnp.dot(z[:, j * hid:(j + 1) * hid], w2pad.T,
                preferred_element_type=jnp.float32)
        for j in range(128 // hid)
    ]
    o_ref[...] = jnp.concatenate(parts, axis=1) * dinv


def _final_body(aggp_ref, hs2p_ref, dinvp_ref, b2t_ref, o_ref):
    o_ref[...] = (aggp_ref[0] + aggp_ref[1] + hs2p_ref[...]) \
        * dinvp_ref[...] + b2t_ref[...]


_SC_PARAMS = pltpu.CompilerParams(use_tc_tiling_on_sc=False,
                                  needs_layout_passes=False)


def _make_deg_kernel(n, cpw):
    mesh = plsc.VectorSubcoreMesh(core_axis_name="c", subcore_axis_name="s")

    @functools.partial(
        pl.kernel,
        out_type=jax.ShapeDtypeStruct((2, n, 32), jnp.float32),
        mesh=mesh,
        compiler_params=_SC_PARAMS,
        scratch_types=[
            pltpu.VMEM((cpw, CHUNK), jnp.int32),
            pltpu.VMEM((CHUNK, 8), jnp.float32),
            pltpu.VMEM((n // 16, 8), jnp.float32),
            pltpu.VMEM((n // 16, 32), jnp.float32),
            pltpu.VMEM_SHARED((n, 8), jnp.float32),
            pltpu.SemaphoreType.DMA,
            pltpu.SemaphoreType.DMA,
            pltpu.SemaphoreType.DMA,
            pltpu.SemaphoreType.DMA,
        ],
    )
    def deg_kernel(ei_hbm, ones_hbm, zeros_hbm, out_hbm, idx_v, ones_v,
                   deg8_v, deg32_v, acc_s, s0, s1, s2, s3):
        c = lax.axis_index("c")
        s = lax.axis_index("s")
        wid = s * 2 + c
        rpt = n // 16
        r0 = s * rpt
        ssems = (s0, s1, s2, s3)
        pltpu.sync_copy(zeros_hbm.at[pl.ds(r0, rpt)], acc_s.at[pl.ds(r0, rpt)])
        pltpu.sync_copy(ones_hbm, ones_v)
        pltpu.sync_copy(ei_hbm.at[1, wid], idx_v)
        plsc.subcore_barrier()

        def body(t, carry):
            for b in range(4):
                j = 4 * t + b

                @pl.when(t > 0)
                def _():
                    pltpu.make_async_copy(
                        ones_v, acc_s.at[idx_v.at[j - 4]], ssems[b]).wait()

                pltpu.async_copy(ones_v, acc_s.at[idx_v.at[j]], ssems[b],
                                 add=True)
            return carry

        lax.fori_loop(0, cpw // 4, body, 0)
        for b in range(4):
            pltpu.make_async_copy(
                ones_v, acc_s.at[idx_v.at[cpw - 4 + b]], ssems[b]).wait()
        plsc.subcore_barrier()
        pltpu.sync_copy(acc_s.at[pl.ds(r0, rpt)], deg8_v)
        zcol = jnp.zeros((16,), jnp.int32)

        def expand(r, carry):
            dv = plsc.load_gather(deg8_v, [jnp.full((16,), r, jnp.int32),
                                           zcol])
            deg32_v[r, pl.ds(0, 16)] = dv
            deg32_v[r, pl.ds(16, 16)] = dv
            return carry

        lax.fori_loop(0, rpt, expand, 0)
        pltpu.sync_copy(deg32_v, out_hbm.at[c, pl.ds(r0, rpt)])

    return deg_kernel


def _make_edge_kernel(n, cpw, d):
    mesh = plsc.VectorSubcoreMesh(core_axis_name="c", subcore_axis_name="s")

    @functools.partial(
        pl.kernel,
        out_type=jax.ShapeDtypeStruct((2, n, d), jnp.float32),
        mesh=mesh,
        compiler_params=_SC_PARAMS,
        scratch_types=(
            [pltpu.VMEM((cpw, CHUNK), jnp.int32),
             pltpu.VMEM((cpw, CHUNK), jnp.int32)]
            + [pltpu.VMEM((CHUNK, d), jnp.float32)] * 8
            + [pltpu.VMEM_SHARED((n, d), jnp.float32)]
            + [pltpu.SemaphoreType.DMA] * 17
        ),
    )
    def edge_kernel(ei_hbm, table_hbm, zeros_hbm, out_hbm,
                    src_v, dst_v, *bufs):
        rows = bufs[0:8]
        acc_s = bufs[8]
        gsems = bufs[9:17]
        ssems = bufs[17:25]
        psem = bufs[25]
        c = lax.axis_index("c")
        s = lax.axis_index("s")
        wid = s * 2 + c
        rpt = n // 16
        rbase = s * rpt
        zslice = zeros_hbm.at[pl.ds(rbase, rpt)]
        aslice = acc_s.at[pl.ds(rbase, rpt)]
        pltpu.async_copy(zslice, aslice, psem)
        pltpu.sync_copy(ei_hbm.at[0, wid], src_v)
        pltpu.sync_copy(ei_hbm.at[1, wid], dst_v)
        for b in range(8):
            pltpu.async_copy(table_hbm.at[src_v.at[b]], rows[b], gsems[b])
        pltpu.make_async_copy(zslice, aslice, psem).wait()
        plsc.subcore_barrier()

        def body(t, carry):
            for b in range(8):
                j = 8 * t + b
                pltpu.make_async_copy(
                    table_hbm.at[src_v.at[j]], rows[b], gsems[b]).wait()
                pltpu.async_copy(rows[b], acc_s.at[dst_v.at[j]], ssems[b],
                                 add=True)
            for b in range(8):
                j = 8 * t + b
                pltpu.make_async_copy(
                    rows[b], acc_s.at[dst_v.at[j]], ssems[b]).wait()

                @pl.when(j + 8 < cpw)
                def _():
                    pltpu.async_copy(
                        table_hbm.at[src_v.at[j + 8]], rows[b], gsems[b])
            return carry

        lax.fori_loop(0, cpw // 8, body, 0)
        plsc.subcore_barrier()
        pltpu.sync_copy(acc_s.at[pl.ds(rbase, rpt)],
                        out_hbm.at[c, pl.ds(rbase, rpt)])

    return edge_kernel


def kernel(x, ei, W_lin, b_lin, W1, b1, W2, b2):
    n, feat = x.shape
    hid = W1.shape[0]
    emb = W2.shape[0]
    e = ei.shape[1]
    cpw = e // (NWORK * CHUNK)
    n_pad = -(-n // 512) * 512
    rh = n_pad * hid // 128

    ei_r = ei.reshape(2, NWORK, cpw, CHUNK)
    ones8 = jnp.ones((CHUNK, 8), jnp.float32)
    zeros8 = jnp.zeros((n_pad, 8), jnp.float32)
    zeros_h = jnp.zeros((n_pad, hid), jnp.float32)

    h1 = pl.pallas_call(
        functools.partial(_enc_body, n_pad=n_pad),
        out_shape=jax.ShapeDtypeStruct((n_pad, hid), jnp.float32),
    )(x, W_lin, b_lin.reshape(1, hid), W1)

    degp = _make_deg_kernel(n_pad, cpw)(ei_r, ones8, zeros8)

    hs1p, dinvp = pl.pallas_call(
        _scale_body,
        out_shape=[
            jax.ShapeDtypeStruct((rh, 128), jnp.float32),
            jax.ShapeDtypeStruct((rh, 128), jnp.float32),
        ],
    )(h1.reshape(rh, 128), degp.reshape(2, rh, 128))

    edge = _make_edge_kernel(n_pad, cpw, hid)
    agg1 = edge(ei_r, hs1p.reshape(n_pad, hid), zeros_h)

    hs2p = pl.pallas_call(
        functools.partial(_layer1_body, hid=hid),
        out_shape=jax.ShapeDtypeStruct((rh, 128), jnp.float32),
    )(agg1.reshape(2, rh, 128), hs1p, dinvp,
      jnp.tile(b1, 128 // hid).reshape(1, 128), W2)

    agg2 = edge(ei_r, hs2p.reshape(n_pad, hid), zeros_h)

    b2t = jnp.tile(jnp.concatenate([b2, jnp.zeros((hid - emb,), jnp.float32)]),
                   128 // hid).reshape(1, 128)
    outp = pl.pallas_call(
        _final_body,
        out_shape=jax.ShapeDtypeStruct((rh, 128), jnp.float32),
    )(agg2.reshape(2, rh, 128), hs2p, dinvp, b2t)

    return outp.reshape(n_pad, hid)[:n, :emb]
